# baseline (speedup 1.0000x reference)
import jax
import jax.numpy as jnp
from jax.experimental import pallas as pl
from jax.experimental.pallas import tpu as pltpu


def _dma_copy(v_ref, o_ref, sem):
    pltpu.make_async_copy(v_ref, o_ref, sem).start()
    pltpu.make_async_copy(v_ref, o_ref, sem).wait()


def kernel(values, indices):
    return pl.pallas_call(
        _dma_copy,
        in_specs=[pl.BlockSpec(memory_space=pltpu.MemorySpace.HBM)],
        out_specs=pl.BlockSpec(memory_space=pltpu.MemorySpace.HBM),
        out_shape=jax.ShapeDtypeStruct(values.shape, values.dtype),
        scratch_shapes=[pltpu.SemaphoreType.DMA],
    )(values)


# speedup vs baseline: 20.9231x; 20.9231x over previous
import jax
import jax.numpy as jnp
from jax.experimental import pallas as pl

_BLOCK = 128 * 1024


def _copy_block(v_ref, o_ref):
    o_ref[...] = v_ref[...]


def kernel(values, indices):
    n = values.shape[0]
    grid = (pl.cdiv(n, _BLOCK),)
    return pl.pallas_call(
        _copy_block,
        grid=grid,
        in_specs=[pl.BlockSpec((_BLOCK,), lambda i: (i,))],
        out_specs=pl.BlockSpec((_BLOCK,), lambda i: (i,)),
        out_shape=jax.ShapeDtypeStruct(values.shape, values.dtype),
    )(values)


# speedup vs baseline: 42.3794x; 2.0255x over previous
import jax
import jax.numpy as jnp
from jax.experimental import pallas as pl

_BLOCK = 1024 * 1024


def _copy_block(v_ref, o_ref):
    o_ref[...] = v_ref[...]


def kernel(values, indices):
    n = values.shape[0]
    grid = (pl.cdiv(n, _BLOCK),)
    return pl.pallas_call(
        _copy_block,
        grid=grid,
        in_specs=[pl.BlockSpec((_BLOCK,), lambda i: (i,))],
        out_specs=pl.BlockSpec((_BLOCK,), lambda i: (i,)),
        out_shape=jax.ShapeDtypeStruct(values.shape, values.dtype),
    )(values)


# speedup vs baseline: 47.4275x; 1.1191x over previous
import jax
import jax.numpy as jnp
from jax.experimental import pallas as pl

_BLOCK = 2048 * 1024


def _copy_block(v_ref, o_ref):
    o_ref[...] = v_ref[...]


def kernel(values, indices):
    n = values.shape[0]
    grid = (pl.cdiv(n, _BLOCK),)
    return pl.pallas_call(
        _copy_block,
        grid=grid,
        in_specs=[pl.BlockSpec((_BLOCK,), lambda i: (i,))],
        out_specs=pl.BlockSpec((_BLOCK,), lambda i: (i,)),
        out_shape=jax.ShapeDtypeStruct(values.shape, values.dtype),
    )(values)
